# baseline (speedup 1.0000x reference)
import jax
import jax.numpy as jnp
from jax.experimental import pallas as pl
from jax.experimental.pallas import tpu as pltpu


def _vq_kernel(z_ref, z2_ref, cb_ref, cb2_ref, zq_ref, idx_ref, loss_ref):
    G = z_ref.shape[0]
    HW = z_ref.shape[2]
    z = jnp.concatenate([z_ref[g] for g in range(G)], axis=1)
    z2 = jnp.concatenate([z2_ref[g] for g in range(G)], axis=1)
    cb = cb_ref[...]
    K = cb.shape[0]
    cb_m2 = cb.astype(jnp.bfloat16) * -2.0
    prod_m2 = jax.lax.dot_general(
        cb_m2, z.astype(jnp.bfloat16), (((1,), (0,)), ((), ())),
        preferred_element_type=jnp.float32)
    d = (z2 + cb2_ref[...]) + prod_m2
    dmin = jnp.min(d, axis=0, keepdims=True)
    iota = jax.lax.broadcasted_iota(jnp.int32, d.shape, 0)
    idx = jnp.min(jnp.where(d == dmin, iota, K), axis=0).astype(jnp.int32)
    onehot = (iota == idx[None, :]).astype(jnp.bfloat16)
    zq = jax.lax.dot_general(
        cb, onehot, (((0,), (0,)), ((), ())),
        preferred_element_type=jnp.float32)
    for g in range(G):
        zq_ref[g] = zq[:, g * HW:(g + 1) * HW]
        idx_ref[g, 0, :] = idx[g * HW:(g + 1) * HW]
    part = jnp.sum((zq - z) ** 2)[None, None]

    @pl.when(pl.program_id(0) == 0)
    def _():
        loss_ref[...] = jnp.zeros_like(loss_ref)

    loss_ref[...] += part


def kernel(z_e, codebook):
    B, C, H, W = z_e.shape
    K, D = codebook.shape
    HW = H * W
    N = B * HW
    zr = z_e.reshape(B, C, HW)
    z2 = z_e[:, 0] ** 2
    for c in range(1, C):
        z2 = z2 + z_e[:, c] ** 2
    z2 = z2.reshape(B, 1, HW)
    cb2 = codebook[:, 0] ** 2
    for c in range(1, D):
        cb2 = cb2 + codebook[:, c] ** 2
    cb2 = cb2[:, None]
    G = 2
    zq, idx, loss_sum = pl.pallas_call(
        _vq_kernel,
        grid=(B // G,),
        in_specs=[
            pl.BlockSpec((G, C, HW), lambda i: (i, 0, 0)),
            pl.BlockSpec((G, 1, HW), lambda i: (i, 0, 0)),
            pl.BlockSpec((K, D), lambda i: (0, 0)),
            pl.BlockSpec((K, 1), lambda i: (0, 0)),
        ],
        out_specs=[
            pl.BlockSpec((G, C, HW), lambda i: (i, 0, 0)),
            pl.BlockSpec((G, 1, HW), lambda i: (i, 0, 0)),
            pl.BlockSpec((1, 1), lambda i: (0, 0)),
        ],
        out_shape=[
            jax.ShapeDtypeStruct((B, C, HW), jnp.float32),
            jax.ShapeDtypeStruct((B, 1, HW), jnp.int32),
            jax.ShapeDtypeStruct((1, 1), jnp.float32),
        ],
    )(zr, z2, codebook, cb2)
    z_q = zq.reshape(B, C, H, W)
    loss = loss_sum[0, 0] * (1.25 / (B * C * H * W))
    return (z_q, loss, idx.reshape(N))


# speedup vs baseline: 1.2640x; 1.2640x over previous
import jax
import jax.numpy as jnp
from jax.experimental import pallas as pl
from jax.experimental.pallas import tpu as pltpu


def _vq_kernel(z_ref, cb_ref, cb2_ref, zq_ref, idx_ref, loss_ref):
    G = z_ref.shape[0]
    HW = z_ref.shape[2]
    z = jnp.concatenate([z_ref[g] for g in range(G)], axis=1)
    cb = cb_ref[...]
    K = cb.shape[0]
    z2 = z[0:1, :] * z[0:1, :]
    for c in range(1, z.shape[0]):
        z2 = z2 + z[c:c+1, :] * z[c:c+1, :]
    cb_m2 = cb.astype(jnp.bfloat16) * -2.0
    prod_m2 = jax.lax.dot_general(
        cb_m2, z.astype(jnp.bfloat16), (((1,), (0,)), ((), ())),
        preferred_element_type=jnp.float32)
    d = (z2 + cb2_ref[...]) + prod_m2
    dmin = jnp.min(d, axis=0, keepdims=True)
    iota = jax.lax.broadcasted_iota(jnp.int32, d.shape, 0)
    idx = jnp.min(jnp.where(d == dmin, iota, K), axis=0).astype(jnp.int32)
    onehot = (iota == idx[None, :]).astype(jnp.bfloat16)
    zq = jax.lax.dot_general(
        cb, onehot, (((0,), (0,)), ((), ())),
        preferred_element_type=jnp.float32)
    for g in range(G):
        zq_ref[g] = zq[:, g * HW:(g + 1) * HW]
        idx_ref[g, 0, :] = idx[g * HW:(g + 1) * HW]
    part = jnp.sum((zq - z) ** 2)[None, None]

    @pl.when(pl.program_id(0) == 0)
    def _():
        loss_ref[...] = jnp.zeros_like(loss_ref)

    loss_ref[...] += part


def kernel(z_e, codebook):
    B, C, H, W = z_e.shape
    K, D = codebook.shape
    HW = H * W
    N = B * HW
    zr = z_e.reshape(B, C, HW)
    cb2 = codebook[:, 0] ** 2
    for c in range(1, D):
        cb2 = cb2 + codebook[:, c] ** 2
    cb2 = cb2[:, None]
    G = 4
    zq, idx, loss_sum = pl.pallas_call(
        _vq_kernel,
        grid=(B // G,),
        in_specs=[
            pl.BlockSpec((G, C, HW), lambda i: (i, 0, 0)),
            pl.BlockSpec((K, D), lambda i: (0, 0)),
            pl.BlockSpec((K, 1), lambda i: (0, 0)),
        ],
        out_specs=[
            pl.BlockSpec((G, C, HW), lambda i: (i, 0, 0)),
            pl.BlockSpec((G, 1, HW), lambda i: (i, 0, 0)),
            pl.BlockSpec((1, 1), lambda i: (0, 0)),
        ],
        out_shape=[
            jax.ShapeDtypeStruct((B, C, HW), jnp.float32),
            jax.ShapeDtypeStruct((B, 1, HW), jnp.int32),
            jax.ShapeDtypeStruct((1, 1), jnp.float32),
        ],
    )(zr, codebook, cb2)
    z_q = zq.reshape(B, C, H, W)
    loss = loss_sum[0, 0] * (1.25 / (B * C * H * W))
    return (z_q, loss, idx.reshape(N))


# speedup vs baseline: 1.2738x; 1.0077x over previous
import jax
import jax.numpy as jnp
from jax.experimental import pallas as pl


def _vq_kernel(z_ref, cb_ref, cb2_ref, zq_ref, idx_ref, loss_ref):
    G = z_ref.shape[0]
    HW = z_ref.shape[2]
    z = jnp.concatenate([z_ref[g] for g in range(G)], axis=1)
    cb = cb_ref[...]
    K = cb.shape[0]
    z2 = z[0:1, :] * z[0:1, :]
    for c in range(1, z.shape[0]):
        z2 = z2 + z[c:c+1, :] * z[c:c+1, :]
    cb_m2 = cb.astype(jnp.bfloat16) * -2.0
    prod_m2 = jax.lax.dot_general(
        cb_m2, z.astype(jnp.bfloat16), (((1,), (0,)), ((), ())),
        preferred_element_type=jnp.float32)
    d = (z2 + cb2_ref[...]) + prod_m2
    dmin = jnp.min(d, axis=0, keepdims=True)
    iota = jax.lax.broadcasted_iota(jnp.int32, d.shape, 0)
    idx = jnp.min(jnp.where(d == dmin, iota, K), axis=0).astype(jnp.int32)
    onehot = (iota == idx[None, :]).astype(jnp.bfloat16)
    zq = jax.lax.dot_general(
        cb, onehot, (((0,), (0,)), ((), ())),
        preferred_element_type=jnp.float32)
    for g in range(G):
        zq_ref[g] = zq[:, g * HW:(g + 1) * HW]
        idx_ref[g, 0, :] = idx[g * HW:(g + 1) * HW]
    part = jnp.sum((zq - z) ** 2)[None, None]

    @pl.when(pl.program_id(0) == 0)
    def _():
        loss_ref[...] = jnp.zeros_like(loss_ref)

    loss_ref[...] += part


def kernel(z_e, codebook):
    B, C, H, W = z_e.shape
    K, D = codebook.shape
    HW = H * W
    N = B * HW
    zr = z_e.reshape(B, C, HW)
    cb2 = codebook[:, 0] ** 2
    for c in range(1, D):
        cb2 = cb2 + codebook[:, c] ** 2
    cb2 = cb2[:, None]
    G = 4
    zq, idx, loss_sum = pl.pallas_call(
        _vq_kernel,
        grid=(B // G,),
        in_specs=[
            pl.BlockSpec((G, C, HW), lambda i: (i, 0, 0)),
            pl.BlockSpec((K, D), lambda i: (0, 0)),
            pl.BlockSpec((K, 1), lambda i: (0, 0)),
        ],
        out_specs=[
            pl.BlockSpec((G, C, HW), lambda i: (i, 0, 0)),
            pl.BlockSpec((G, 1, HW), lambda i: (i, 0, 0)),
            pl.BlockSpec((1, 1), lambda i: (0, 0)),
        ],
        out_shape=[
            jax.ShapeDtypeStruct((B, C, HW), jnp.float32),
            jax.ShapeDtypeStruct((B, 1, HW), jnp.int32),
            jax.ShapeDtypeStruct((1, 1), jnp.float32),
        ],
    )(zr, codebook, cb2)
    z_q = zq.reshape(B, C, H, W)
    loss = loss_sum[0, 0] * (1.25 / (B * C * H * W))
    return (z_q, loss, idx.reshape(N))
